# baseline (speedup 1.0000x reference)
import functools

import jax
import jax.numpy as jnp
from jax import lax
from jax.experimental import pallas as pl
from jax.experimental.pallas import tpu as pltpu
from jax.experimental.pallas import tpu_sc as plsc

OUT_DIM = 128
BATCH = 1024
SEQ = 300

NUM_CORES = 2
NUM_SUBCORES = 16
NW = NUM_CORES * NUM_SUBCORES

B = BATCH * SEQ
B_PER_W = B // NW
CHUNK = 128
NCH = B_PER_W // CHUNK
NPAIR = NCH // 2


def _gather_rows(ids_flat, table):
    mesh = plsc.VectorSubcoreMesh(
        core_axis_name="c", subcore_axis_name="s",
        num_cores=NUM_CORES, num_subcores=NUM_SUBCORES)

    @functools.partial(
        pl.kernel,
        out_type=jax.ShapeDtypeStruct((B, OUT_DIM), jnp.float32),
        mesh=mesh,
        scratch_types=[
            pltpu.VMEM((B_PER_W,), jnp.int32),
            pltpu.VMEM((CHUNK, OUT_DIM), jnp.float32),
            pltpu.VMEM((CHUNK, OUT_DIM), jnp.float32),
            pltpu.SemaphoreType.DMA,
            pltpu.SemaphoreType.DMA,
        ],
    )
    def k(ids_hbm, table_hbm, out_hbm, idx_v, buf0, buf1, sem0, sem1):
        wid = lax.axis_index("s") * NUM_CORES + lax.axis_index("c")
        base = pl.multiple_of(wid * B_PER_W, CHUNK)
        pltpu.sync_copy(ids_hbm.at[pl.ds(base, B_PER_W)], idx_v)

        def start_gather(c, buf, sem):
            off = pl.multiple_of(c * CHUNK, CHUNK)
            pltpu.async_copy(table_hbm.at[idx_v.at[pl.ds(off, CHUNK)]], buf, sem)

        def wait_gather(buf, sem):
            pltpu.make_async_copy(table_hbm.at[pl.ds(0, CHUNK)], buf, sem).wait()

        def store(c, buf):
            off = pl.multiple_of(base + c * CHUNK, CHUNK)
            pltpu.sync_copy(buf, out_hbm.at[pl.ds(off, CHUNK)])

        start_gather(0, buf0, sem0)

        @pl.loop(0, NPAIR)
        def _(i):
            c0 = 2 * i
            start_gather(c0 + 1, buf1, sem1)
            wait_gather(buf0, sem0)
            store(c0, buf0)
            start_gather(c0 + 2, buf0, sem0)
            wait_gather(buf1, sem1)
            store(c0 + 1, buf1)

        wait_gather(buf0, sem0)
        store(NCH - 1, buf0)

    return k(ids_flat, table)


_COPY_ROWS = 2048


def _tc_copy(x):
    def body(x_ref, o_ref):
        o_ref[...] = x_ref[...]

    return pl.pallas_call(
        body,
        grid=(B // _COPY_ROWS,),
        in_specs=[pl.BlockSpec((_COPY_ROWS, OUT_DIM), lambda i: (i, 0))],
        out_specs=pl.BlockSpec((_COPY_ROWS, OUT_DIM), lambda i: (i, 0)),
        out_shape=jax.ShapeDtypeStruct((B, OUT_DIM), jnp.float32),
    )(x)


def kernel(input_ids, attention_mask, W):
    ids_flat = input_ids.reshape(-1)
    e1 = _gather_rows(ids_flat, W)
    e2 = _tc_copy(e1)
    e1 = e1.reshape(BATCH, SEQ, OUT_DIM)
    e2 = e2.reshape(BATCH, SEQ, OUT_DIM)
    return (e1, e2, attention_mask)


# speedup vs baseline: 1.5625x; 1.5625x over previous
import functools

import jax
import jax.numpy as jnp
from jax import lax
from jax.experimental import pallas as pl
from jax.experimental.pallas import tpu as pltpu
from jax.experimental.pallas import tpu_sc as plsc

OUT_DIM = 128
BATCH = 1024
SEQ = 300

NUM_CORES = 2
NUM_SUBCORES = 16
NW = NUM_CORES * NUM_SUBCORES

B = BATCH * SEQ
B_PER_W = B // NW
CHUNK = 128
NCH = B_PER_W // CHUNK
NPAIR = NCH // 2


def _gather_rows(ids_flat, table):
    mesh = plsc.VectorSubcoreMesh(
        core_axis_name="c", subcore_axis_name="s",
        num_cores=NUM_CORES, num_subcores=NUM_SUBCORES)

    @functools.partial(
        pl.kernel,
        out_type=jax.ShapeDtypeStruct((B, OUT_DIM), jnp.float32),
        mesh=mesh,
        scratch_types=[
            pltpu.VMEM((B_PER_W,), jnp.int32),
            pltpu.VMEM((CHUNK, OUT_DIM), jnp.float32),
            pltpu.VMEM((CHUNK, OUT_DIM), jnp.float32),
            pltpu.SemaphoreType.DMA,
            pltpu.SemaphoreType.DMA,
        ],
    )
    def k(ids_hbm, table_hbm, out_hbm, idx_v, buf0, buf1, sem0, sem1):
        wid = lax.axis_index("s") * NUM_CORES + lax.axis_index("c")
        base = pl.multiple_of(wid * B_PER_W, CHUNK)
        pltpu.sync_copy(ids_hbm.at[pl.ds(base, B_PER_W)], idx_v)

        def start_gather(c, buf, sem):
            off = pl.multiple_of(c * CHUNK, CHUNK)
            pltpu.async_copy(table_hbm.at[idx_v.at[pl.ds(off, CHUNK)]], buf, sem)

        def wait_gather(buf, sem):
            pltpu.make_async_copy(table_hbm.at[pl.ds(0, CHUNK)], buf, sem).wait()

        def store(c, buf):
            off = pl.multiple_of(base + c * CHUNK, CHUNK)
            pltpu.sync_copy(buf, out_hbm.at[pl.ds(off, CHUNK)])

        start_gather(0, buf0, sem0)

        @pl.loop(0, NPAIR)
        def _(i):
            c0 = 2 * i
            start_gather(c0 + 1, buf1, sem1)
            wait_gather(buf0, sem0)
            store(c0, buf0)
            start_gather(c0 + 2, buf0, sem0)
            wait_gather(buf1, sem1)
            store(c0 + 1, buf1)

        wait_gather(buf0, sem0)
        store(NCH - 1, buf0)

    return k(ids_flat, table)


_BB = 16


def _reshape_dup(flat):
    def body(x_ref, o1_ref, o2_ref):
        x = x_ref[...].reshape(_BB, SEQ, OUT_DIM)
        o1_ref[...] = x
        o2_ref[...] = x

    out_sds = jax.ShapeDtypeStruct((BATCH, SEQ, OUT_DIM), jnp.float32)
    return pl.pallas_call(
        body,
        grid=(BATCH // _BB,),
        in_specs=[pl.BlockSpec((_BB * SEQ, OUT_DIM), lambda i: (i, 0))],
        out_specs=(pl.BlockSpec((_BB, SEQ, OUT_DIM), lambda i: (i, 0, 0)),
                   pl.BlockSpec((_BB, SEQ, OUT_DIM), lambda i: (i, 0, 0))),
        out_shape=(out_sds, out_sds),
    )(flat)


def kernel(input_ids, attention_mask, W):
    ids_flat = input_ids.reshape(-1)
    flat = _gather_rows(ids_flat, W)
    e1, e2 = _reshape_dup(flat)
    return (e1, e2, attention_mask)


# speedup vs baseline: 3.8949x; 2.4927x over previous
import functools

import jax
import jax.numpy as jnp
from jax import lax
from jax.experimental import pallas as pl
from jax.experimental.pallas import tpu as pltpu
from jax.experimental.pallas import tpu_sc as plsc

OUT_DIM = 128
BATCH = 1024
SEQ = 300

NUM_CORES = 2
NUM_SUBCORES = 16
NW = NUM_CORES * NUM_SUBCORES

B = BATCH * SEQ
B_PER_W = B // NW
CHUNK = 128
NCH = B_PER_W // CHUNK
NPAIR = NCH // 2


def _gather_rows2(ids_flat, table):
    mesh = plsc.VectorSubcoreMesh(
        core_axis_name="c", subcore_axis_name="s",
        num_cores=NUM_CORES, num_subcores=NUM_SUBCORES)

    out_sds = jax.ShapeDtypeStruct((B, OUT_DIM), jnp.float32)

    @functools.partial(
        pl.kernel,
        out_type=(out_sds, out_sds),
        mesh=mesh,
        scratch_types=[
            pltpu.VMEM((B_PER_W,), jnp.int32),
            pltpu.VMEM((CHUNK, OUT_DIM), jnp.float32),
            pltpu.VMEM((CHUNK, OUT_DIM), jnp.float32),
            pltpu.SemaphoreType.DMA,
            pltpu.SemaphoreType.DMA,
        ],
    )
    def k(ids_hbm, table_hbm, out_hbm, out2_hbm, idx_v, buf0, buf1, sem0, sem1):
        wid = lax.axis_index("s") * NUM_CORES + lax.axis_index("c")
        base = pl.multiple_of(wid * B_PER_W, CHUNK)
        pltpu.sync_copy(ids_hbm.at[pl.ds(base, B_PER_W)], idx_v)

        def start_gather(c, buf, sem):
            off = pl.multiple_of(c * CHUNK, CHUNK)
            pltpu.async_copy(table_hbm.at[idx_v.at[pl.ds(off, CHUNK)]], buf, sem)

        def wait_gather(buf, sem):
            pltpu.make_async_copy(table_hbm.at[pl.ds(0, CHUNK)], buf, sem).wait()

        def store(c, buf):
            off = pl.multiple_of(base + c * CHUNK, CHUNK)
            pltpu.sync_copy(buf, out_hbm.at[pl.ds(off, CHUNK)])
            pltpu.sync_copy(buf, out2_hbm.at[pl.ds(off, CHUNK)])

        start_gather(0, buf0, sem0)

        @pl.loop(0, NPAIR)
        def _(i):
            c0 = 2 * i
            start_gather(c0 + 1, buf1, sem1)
            wait_gather(buf0, sem0)
            store(c0, buf0)
            start_gather(c0 + 2, buf0, sem0)
            wait_gather(buf1, sem1)
            store(c0 + 1, buf1)

        wait_gather(buf0, sem0)
        store(NCH - 1, buf0)

    return k(ids_flat, table)


def kernel(input_ids, attention_mask, W):
    ids_sm = input_ids.T.reshape(-1)
    f1, f2 = _gather_rows2(ids_sm, W)
    e1 = f1.reshape(SEQ, BATCH, OUT_DIM).transpose(1, 0, 2)
    e2 = f2.reshape(SEQ, BATCH, OUT_DIM).transpose(1, 0, 2)
    return (e1, e2, attention_mask)
